# baseline (speedup 1.0000x reference)
import jax
import jax.numpy as jnp
from jax import lax
from jax.experimental import pallas as pl
from jax.experimental.pallas import tpu as pltpu
from jax.experimental.pallas import tpu_sc as plsc

B, L, D = 16, 2048, 512
LANES = 16
CHUNKS = L // LANES


def _body(t32_hbm, seq_hbm, tout_hbm, sout_hbm, t4_hbm, trow, srow, t4v):
    s = lax.axis_index("s")

    @pl.when(s < B)
    def _():
        b = s
        pltpu.sync_copy(t32_hbm.at[b], trow)

        def count_chunk(i, acc):
            v = trow[pl.ds(i * LANES, LANES)]
            return acc + (v != 0).astype(jnp.int32)

        acc = lax.fori_loop(jnp.int32(0), jnp.int32(CHUNKS), count_chunk,
                            jnp.zeros((LANES,), jnp.int32))
        seq_len = jnp.sum(acc, dtype=jnp.int32)
        last = seq_len - 1

        base = (last // LANES) * LANES
        off = last - base
        chunk_v = trow[pl.ds(base, LANES)]
        lane = lax.iota(jnp.int32, LANES)
        trow[pl.ds(base, LANES)] = jnp.where(lane == off, 0, chunk_v)

        pltpu.sync_copy(trow, tout_hbm.at[b])
        pltpu.sync_copy(seq_hbm.at[b, pl.ds(last, 1)], srow)
        pltpu.sync_copy(srow, sout_hbm.at[b])

    @pl.when(s == 0)
    def _():
        t4v[...] = jnp.full((LANES,), 1.0, jnp.float32)
        pltpu.sync_copy(t4v, t4_hbm)


_mesh = plsc.VectorSubcoreMesh(core_axis_name="c", subcore_axis_name="s",
                               num_cores=1, num_subcores=16)

_sc_call = pl.kernel(
    _body,
    out_type=(
        jax.ShapeDtypeStruct((B, L), jnp.int32),
        jax.ShapeDtypeStruct((B, 1, D), jnp.float32),
        jax.ShapeDtypeStruct((B,), jnp.float32),
    ),
    mesh=_mesh,
    scratch_types=[
        pltpu.VMEM((L,), jnp.int32),
        pltpu.VMEM((1, D), jnp.float32),
        pltpu.VMEM((LANES,), jnp.float32),
    ],
    compiler_params=pltpu.CompilerParams(needs_layout_passes=False),
)


_ROWS_A = 3


def _copy_body(x_ref, o_ref):
    o_ref[...] = x_ref[...]


def _copy_tail_body(x_ref, a_ref, o_ref):
    del a_ref
    o_ref[...] = x_ref[...]


def _tc_copy_head(x):
    return pl.pallas_call(
        _copy_body,
        out_shape=jax.ShapeDtypeStruct((B, L, D), jnp.float32),
        grid=(_ROWS_A,),
        in_specs=[pl.BlockSpec(
            (1, L, D), lambda i: (i, jnp.int32(0), jnp.int32(0)))],
        out_specs=pl.BlockSpec(
            (1, L, D), lambda i: (i, jnp.int32(0), jnp.int32(0))),
    )(x)


def _tc_copy_tail(x, acc):
    return pl.pallas_call(
        _copy_tail_body,
        out_shape=jax.ShapeDtypeStruct((B, L, D), jnp.float32),
        grid=(B - _ROWS_A,),
        in_specs=[
            pl.BlockSpec(
                (1, L, D),
                lambda i: (i + jnp.int32(_ROWS_A), jnp.int32(0),
                           jnp.int32(0))),
            pl.BlockSpec(memory_space=pl.ANY),
        ],
        out_specs=pl.BlockSpec(
            (1, L, D),
            lambda i: (i + jnp.int32(_ROWS_A), jnp.int32(0), jnp.int32(0))),
        input_output_aliases={1: 0},
    )(x, acc)


def kernel(seq3, time3):
    t32 = time3.astype(jnp.int32)
    head = _tc_copy_head(seq3)
    t32b, head_b = lax.optimization_barrier((t32, head))
    tout, seq4, t4 = _sc_call(t32b, seq3)
    seq3_out = _tc_copy_tail(seq3, head_b)
    time3_new = tout.astype(jnp.uint32).astype(time3.dtype)
    time4 = t4[:, None]
    return (seq3_out, time3_new, seq4, time4)


# speedup vs baseline: 1.0304x; 1.0304x over previous
import jax
import jax.numpy as jnp
from jax import lax
from jax.experimental import pallas as pl
from jax.experimental.pallas import tpu as pltpu
from jax.experimental.pallas import tpu_sc as plsc

B, L, D = 16, 2048, 512
LANES = 16
CHUNKS = L // LANES


def _body(t32_hbm, seq_hbm, tout_hbm, sout_hbm, t4_hbm, trow, srow, t4v):
    s = lax.axis_index("s")

    @pl.when(s < B)
    def _():
        b = s
        pltpu.sync_copy(t32_hbm.at[b], trow)

        def count_chunk(i, acc):
            v = trow[pl.ds(i * LANES, LANES)]
            return acc + (v != 0).astype(jnp.int32)

        acc = lax.fori_loop(jnp.int32(0), jnp.int32(CHUNKS), count_chunk,
                            jnp.zeros((LANES,), jnp.int32))
        seq_len = jnp.sum(acc, dtype=jnp.int32)
        last = seq_len - 1

        base = (last // LANES) * LANES
        off = last - base
        chunk_v = trow[pl.ds(base, LANES)]
        lane = lax.iota(jnp.int32, LANES)
        trow[pl.ds(base, LANES)] = jnp.where(lane == off, 0, chunk_v)

        pltpu.sync_copy(trow, tout_hbm.at[b])
        pltpu.sync_copy(seq_hbm.at[b, pl.ds(last, 1)], srow)
        pltpu.sync_copy(srow, sout_hbm.at[b])

    @pl.when(s == 0)
    def _():
        t4v[...] = jnp.full((LANES,), 1.0, jnp.float32)
        pltpu.sync_copy(t4v, t4_hbm)


_mesh = plsc.VectorSubcoreMesh(core_axis_name="c", subcore_axis_name="s",
                               num_cores=1, num_subcores=16)

_sc_call = pl.kernel(
    _body,
    out_type=(
        jax.ShapeDtypeStruct((B, L), jnp.int32),
        jax.ShapeDtypeStruct((B, 1, D), jnp.float32),
        jax.ShapeDtypeStruct((B,), jnp.float32),
    ),
    mesh=_mesh,
    scratch_types=[
        pltpu.VMEM((L,), jnp.int32),
        pltpu.VMEM((1, D), jnp.float32),
        pltpu.VMEM((LANES,), jnp.float32),
    ],
    compiler_params=pltpu.CompilerParams(needs_layout_passes=False),
)


def _copy_body(x_ref, o_ref):
    o_ref[...] = x_ref[...]


def _tc_copy(x):
    return pl.pallas_call(
        _copy_body,
        out_shape=jax.ShapeDtypeStruct((B, L, D), jnp.float32),
        grid=(B,),
        in_specs=[pl.BlockSpec(
            (1, L, D), lambda i: (i, jnp.int32(0), jnp.int32(0)))],
        out_specs=pl.BlockSpec(
            (1, L, D), lambda i: (i, jnp.int32(0), jnp.int32(0))),
    )(x)


def kernel(seq3, time3):
    t32 = time3.astype(jnp.int32)
    tout, seq4, t4 = _sc_call(t32, seq3)
    seq3_out = _tc_copy(seq3)
    time3_new = tout.astype(jnp.uint32).astype(time3.dtype)
    time4 = t4[:, None]
    return (seq3_out, time3_new, seq4, time4)
